# baseline (speedup 1.0000x reference)
import functools

import jax
import jax.numpy as jnp
from jax import lax
from jax.experimental import pallas as pl
from jax.experimental.pallas import tpu as pltpu
from jax.experimental.pallas import tpu_sc as plsc

BATCH = 1024
MAX_LEN = 200
D_MODEL = 128
VOCAB = 100000

NUM_CORES = 2
NUM_SUBCORES = 16
NW = NUM_CORES * NUM_SUBCORES
ROWS_PER_W = (BATCH * MAX_LEN) // NW
CHUNK = 128
NCHUNK = ROWS_PER_W // CHUNK
NBUF = 5


def _pe_body(pe_ref):
    pos = lax.broadcasted_iota(jnp.int32, (MAX_LEN, D_MODEL), 0).astype(
        jnp.float32)
    d = lax.broadcasted_iota(jnp.int32, (MAX_LEN, D_MODEL), 1)
    even_i = ((d // 2) * 2).astype(jnp.float32)
    inv_denom = jnp.reciprocal(jnp.power(10000.0, even_i / D_MODEL))
    angle = pos * inv_denom
    pe_ref[...] = jnp.where(d % 2 == 0, jnp.sin(angle), jnp.cos(angle))


def _compute_pe():
    return pl.pallas_call(
        _pe_body,
        out_shape=jax.ShapeDtypeStruct((MAX_LEN, D_MODEL), jnp.float32),
    )()


def _sc_body(tok_hbm, table_hbm, pe_hbm, out_hbm, idx_v, pe_v, rows,
             sin_, sout):
    wid = lax.axis_index("s") * NUM_CORES + lax.axis_index("c")
    base = wid * ROWS_PER_W
    pltpu.sync_copy(tok_hbm.at[pl.ds(base, ROWS_PER_W)], idx_v)
    pltpu.sync_copy(pe_hbm, pe_v)

    def issue_gather(s, b):
        pltpu.async_copy(table_hbm.at[idx_v.at[pl.ds(s * CHUNK, CHUNK)]],
                         rows[b], sin_[b])

    def wait_gather(b):
        pltpu.make_async_copy(out_hbm.at[pl.ds(0, CHUNK)], rows[b],
                              sin_[b]).wait()

    def wait_out(b):
        pltpu.make_async_copy(rows[b], out_hbm.at[pl.ds(0, CHUNK)],
                              sout[b]).wait()

    for b in range(NBUF - 1):
        issue_gather(b, b)

    @pl.loop(0, NCHUNK, step=NBUF)
    def _blk(s0):
        for b in range(NBUF):
            s = s0 + b
            wait_gather(b)

            phase = lax.rem(s * CHUNK, MAX_LEN)
            split = jnp.minimum(MAX_LEN - phase, CHUNK)

            @pl.loop(0, split)
            def _r1(r):
                for c in range(D_MODEL // 16):
                    plsc.addupdate(rows[b].at[r, pl.ds(c * 16, 16)],
                                   pe_v[phase + r, pl.ds(c * 16, 16)])

            @pl.loop(split, CHUNK)
            def _r2(r):
                for c in range(D_MODEL // 16):
                    plsc.addupdate(rows[b].at[r, pl.ds(c * 16, 16)],
                                   pe_v[phase + r - MAX_LEN,
                                        pl.ds(c * 16, 16)])

            pltpu.async_copy(rows[b],
                             out_hbm.at[pl.ds(base + s * CHUNK, CHUNK)],
                             sout[b])

            @pl.when(s >= 1)
            def _():
                wait_out((b - 1) % NBUF)

            @pl.when(s + NBUF - 1 < NCHUNK)
            def _():
                issue_gather(s + NBUF - 1, (b + NBUF - 1) % NBUF)

    wait_out((NCHUNK - 1) % NBUF)


@functools.partial(
    pl.kernel,
    out_type=jax.ShapeDtypeStruct((BATCH * MAX_LEN, D_MODEL), jnp.float32),
    mesh=plsc.VectorSubcoreMesh(core_axis_name="c", subcore_axis_name="s",
                                num_cores=NUM_CORES,
                                num_subcores=NUM_SUBCORES),
    scratch_types=[
        pltpu.VMEM((ROWS_PER_W,), jnp.int32),
        pltpu.VMEM((MAX_LEN, D_MODEL), jnp.float32),
        [pltpu.VMEM((CHUNK, D_MODEL), jnp.float32) for _ in range(NBUF)],
        [pltpu.SemaphoreType.DMA for _ in range(NBUF)],
        [pltpu.SemaphoreType.DMA for _ in range(NBUF)],
    ],
)
def _sc_embed(tok_hbm, table_hbm, pe_hbm, out_hbm, idx_v, pe_v, rows,
              sin_, sout):
    _sc_body(tok_hbm, table_hbm, pe_hbm, out_hbm, idx_v, pe_v, rows,
             sin_, sout)


@jax.jit
def kernel(token_ids, emb_table):
    pe = _compute_pe()
    flat = token_ids.reshape(BATCH * MAX_LEN)
    out = _sc_embed(flat, emb_table, pe)
    return out.reshape(BATCH, MAX_LEN, D_MODEL)


# speedup vs baseline: 2.0129x; 2.0129x over previous
import functools

import jax
import jax.numpy as jnp
from jax import lax
from jax.experimental import pallas as pl
from jax.experimental.pallas import tpu as pltpu
from jax.experimental.pallas import tpu_sc as plsc

BATCH = 1024
MAX_LEN = 200
D_MODEL = 128
VOCAB = 100000

NUM_CORES = 2
NUM_SUBCORES = 16
NW = NUM_CORES * NUM_SUBCORES
ROWS_PER_W = (BATCH * MAX_LEN) // NW
SENT_PER_W = ROWS_PER_W // MAX_LEN


def _pe_body(pe_ref):
    pos = lax.broadcasted_iota(jnp.int32, (MAX_LEN, D_MODEL), 0).astype(
        jnp.float32)
    d = lax.broadcasted_iota(jnp.int32, (MAX_LEN, D_MODEL), 1)
    even_i = ((d // 2) * 2).astype(jnp.float32)
    inv_denom = jnp.reciprocal(jnp.power(10000.0, even_i / D_MODEL))
    angle = pos * inv_denom
    pe_ref[...] = jnp.where(d % 2 == 0, jnp.sin(angle), jnp.cos(angle))


def _compute_pe():
    return pl.pallas_call(
        _pe_body,
        out_shape=jax.ShapeDtypeStruct((MAX_LEN, D_MODEL), jnp.float32),
    )()


def _sc_body(tok_hbm, table_hbm, pe_hbm, out_hbm, idx_v, pe_v, rows,
             sin_, sout):
    wid = lax.axis_index("s") * NUM_CORES + lax.axis_index("c")
    base = wid * ROWS_PER_W
    pltpu.sync_copy(tok_hbm.at[pl.ds(base, ROWS_PER_W)], idx_v)
    pltpu.sync_copy(pe_hbm, pe_v)

    def issue_gather(s, b):
        pltpu.async_copy(table_hbm.at[idx_v.at[pl.ds(s * MAX_LEN, 128)]],
                         rows[b].at[pl.ds(0, 128)], sin_[b])
        pltpu.async_copy(
            table_hbm.at[idx_v.at[pl.ds(s * MAX_LEN + 128, MAX_LEN - 128)]],
            rows[b].at[pl.ds(128, MAX_LEN - 128)], sin_[b])

    def wait_gather(b):
        pltpu.make_async_copy(out_hbm.at[pl.ds(0, MAX_LEN)], rows[b],
                              sin_[b]).wait()

    def wait_out(b):
        pltpu.make_async_copy(rows[b], out_hbm.at[pl.ds(0, MAX_LEN)],
                              sout[b]).wait()

    issue_gather(0, 0)

    @pl.loop(0, SENT_PER_W, step=2)
    def _blk(s0):
        for b in range(2):
            s = s0 + b
            wait_gather(b)

            @pl.when(s >= 1)
            def _():
                wait_out(1 - b)

            @pl.when(s + 1 < SENT_PER_W)
            def _():
                issue_gather(s + 1, 1 - b)

            @pl.loop(0, MAX_LEN)
            def _row(r):
                for c in range(D_MODEL // 16):
                    plsc.addupdate(rows[b].at[r, pl.ds(c * 16, 16)],
                                   pe_v[r, pl.ds(c * 16, 16)])

            pltpu.async_copy(rows[b],
                             out_hbm.at[pl.ds(base + s * MAX_LEN, MAX_LEN)],
                             sout[b])

    wait_out((SENT_PER_W - 1) % 2)


@functools.partial(
    pl.kernel,
    out_type=jax.ShapeDtypeStruct((BATCH * MAX_LEN, D_MODEL), jnp.float32),
    mesh=plsc.VectorSubcoreMesh(core_axis_name="c", subcore_axis_name="s",
                                num_cores=NUM_CORES,
                                num_subcores=NUM_SUBCORES),
    scratch_types=[
        pltpu.VMEM((ROWS_PER_W,), jnp.int32),
        pltpu.VMEM((MAX_LEN, D_MODEL), jnp.float32),
        [pltpu.VMEM((MAX_LEN, D_MODEL), jnp.float32) for _ in range(2)],
        [pltpu.SemaphoreType.DMA for _ in range(2)],
        [pltpu.SemaphoreType.DMA for _ in range(2)],
    ],
)
def _sc_embed(tok_hbm, table_hbm, pe_hbm, out_hbm, idx_v, pe_v, rows,
              sin_, sout):
    _sc_body(tok_hbm, table_hbm, pe_hbm, out_hbm, idx_v, pe_v, rows,
             sin_, sout)


@jax.jit
def kernel(token_ids, emb_table):
    pe = _compute_pe()
    flat = token_ids.reshape(BATCH * MAX_LEN)
    out = _sc_embed(flat, emb_table, pe)
    return out.reshape(BATCH, MAX_LEN, D_MODEL)


# speedup vs baseline: 2.3024x; 1.1438x over previous
import functools

import jax
import jax.numpy as jnp
from jax import lax
from jax.experimental import pallas as pl
from jax.experimental.pallas import tpu as pltpu
from jax.experimental.pallas import tpu_sc as plsc

BATCH = 1024
MAX_LEN = 200
D_MODEL = 128
VOCAB = 100000

NUM_CORES = 2
NUM_SUBCORES = 16
NW = NUM_CORES * NUM_SUBCORES
ROWS_PER_W = (BATCH * MAX_LEN) // NW
SENT_PER_W = ROWS_PER_W // MAX_LEN


def _pe_body(pe_ref):
    pos = lax.broadcasted_iota(jnp.int32, (MAX_LEN, D_MODEL), 0).astype(
        jnp.float32)
    d = lax.broadcasted_iota(jnp.int32, (MAX_LEN, D_MODEL), 1)
    even_i = ((d // 2) * 2).astype(jnp.float32)
    inv_denom = jnp.reciprocal(jnp.power(10000.0, even_i / D_MODEL))
    angle = pos * inv_denom
    pe_ref[...] = jnp.where(d % 2 == 0, jnp.sin(angle), jnp.cos(angle))


def _compute_pe():
    return pl.pallas_call(
        _pe_body,
        out_shape=jax.ShapeDtypeStruct((MAX_LEN, D_MODEL), jnp.float32),
    )()


def _sc_body(tok_hbm, table_hbm, pe_hbm, out_hbm, idx_v, pe_v, rows,
             sin_, sout):
    wid = lax.axis_index("s") * NUM_CORES + lax.axis_index("c")
    base = wid * ROWS_PER_W
    pltpu.sync_copy(tok_hbm.at[pl.ds(base, ROWS_PER_W)], idx_v)
    pltpu.sync_copy(pe_hbm, pe_v)

    def issue_gather(s, b):
        pltpu.async_copy(table_hbm.at[idx_v.at[pl.ds(s * MAX_LEN, 128)]],
                         rows[b].at[pl.ds(0, 128)], sin_[b])
        pltpu.async_copy(
            table_hbm.at[idx_v.at[pl.ds(s * MAX_LEN + 128, MAX_LEN - 128)]],
            rows[b].at[pl.ds(128, MAX_LEN - 128)], sin_[b])

    def wait_gather(b):
        pltpu.make_async_copy(out_hbm.at[pl.ds(0, MAX_LEN)], rows[b],
                              sin_[b]).wait()

    def wait_out(b):
        pltpu.make_async_copy(rows[b], out_hbm.at[pl.ds(0, MAX_LEN)],
                              sout[b]).wait()

    def add_pe(b):
        @pl.loop(0, MAX_LEN)
        def _row(r):
            for c in range(D_MODEL // 16):
                plsc.addupdate(rows[b].at[r, pl.ds(c * 16, 16)],
                               pe_v[r, pl.ds(c * 16, 16)])

    def issue_out(s, b):
        pltpu.async_copy(rows[b],
                         out_hbm.at[pl.ds(base + s * MAX_LEN, MAX_LEN)],
                         sout[b])

    issue_gather(0, 0)
    issue_gather(1, 1)

    @pl.loop(0, SENT_PER_W - 2, step=3)
    def _blk(s0):
        for b in range(3):
            s = s0 + b
            wait_gather(b)
            add_pe(b)
            issue_out(s, b)
            if b == 0:
                @pl.when(s0 >= 1)
                def _():
                    wait_out((b - 1) % 3)
            else:
                wait_out(b - 1)
            issue_gather(s + 2, (b + 2) % 3)

    for s in (SENT_PER_W - 2, SENT_PER_W - 1):
        b = s % 3
        wait_gather(b)
        add_pe(b)
        issue_out(s, b)
    for b in range(3):
        wait_out(b)


@functools.partial(
    pl.kernel,
    out_type=jax.ShapeDtypeStruct((BATCH * MAX_LEN, D_MODEL), jnp.float32),
    mesh=plsc.VectorSubcoreMesh(core_axis_name="c", subcore_axis_name="s",
                                num_cores=NUM_CORES,
                                num_subcores=NUM_SUBCORES),
    scratch_types=[
        pltpu.VMEM((ROWS_PER_W,), jnp.int32),
        pltpu.VMEM((MAX_LEN, D_MODEL), jnp.float32),
        [pltpu.VMEM((MAX_LEN, D_MODEL), jnp.float32) for _ in range(3)],
        [pltpu.SemaphoreType.DMA for _ in range(3)],
        [pltpu.SemaphoreType.DMA for _ in range(3)],
    ],
)
def _sc_embed(tok_hbm, table_hbm, pe_hbm, out_hbm, idx_v, pe_v, rows,
              sin_, sout):
    _sc_body(tok_hbm, table_hbm, pe_hbm, out_hbm, idx_v, pe_v, rows,
             sin_, sout)


@jax.jit
def kernel(token_ids, emb_table):
    pe = _compute_pe()
    flat = token_ids.reshape(BATCH * MAX_LEN)
    out = _sc_embed(flat, emb_table, pe)
    return out.reshape(BATCH, MAX_LEN, D_MODEL)
